# baseline (speedup 1.0000x reference)
import functools

import jax
import jax.numpy as jnp
from jax import lax
from jax.experimental import pallas as pl
from jax.experimental.pallas import tpu as pltpu
from jax.experimental.pallas import tpu_sc as plsc

_DIM = 32
_BATCH = 16384
_NC = 2
_NS = 16
_L = 16
_NW = _NC * _NS
_BPW = _BATCH // _NW
_ROWW = 128
_CHUNK = 128
_NCHUNKS = _BPW // _CHUNK


def _sc_kernel(wt_hbm, idx_hbm, out_hbm, idx_v, ridx_v, tblT, gbuf, cbuf,
               obuf, shared, sem, lsem):
  sid = lax.axis_index("s")
  wid = sid * _NC + lax.axis_index("c")
  base = pl.multiple_of(wid * _BPW, _BPW)

  pltpu.sync_copy(idx_hbm.at[pl.ds(base, _BPW)], idx_v)

  lane = lax.iota(jnp.int32, _L)

  def shift(g, _):
    off = pl.multiple_of(g * _L, _L)
    v = idx_v[pl.ds(off, _L)]
    ridx_v[pl.ds(off, _L)] = v >> 2
    cpos = (lane + g * _L) & (_CHUNK - 1)
    base = cpos * _ROWW + (v & 3) * _DIM
    for k in range(_L):
      tblT[pl.ds(k * _BPW + g * _L, _L)] = base + 2 * k
    return 0

  lax.fori_loop(0, _BPW // _L, shift, 0)

  copies = []
  for j in range(_NCHUNKS):
    copies.append(
        pltpu.async_copy(
            wt_hbm.at[ridx_v.at[pl.ds(j * _CHUNK, _CHUNK)]],
            gbuf.at[pl.ds(j * _CHUNK, _CHUNK)],
            sem.at[j],
        )
    )

  kstride = lax.iota(jnp.int32, _L) * _BPW

  for j in range(_NCHUNKS):
    copies[j].wait()

    def restage(r, _):
      pltpu.sync_copy(gbuf.at[j * _CHUNK + r],
                      shared.at[pl.ds(sid * _CHUNK * _ROWW + r * _ROWW, _ROWW)])
      return 0

    lax.fori_loop(0, _CHUNK, restage, 0)
    pltpu.sync_copy(shared.at[pl.ds(sid * _CHUNK * _ROWW, _CHUNK * _ROWW)], cbuf)

    def convert_one(c, _):
      av = plsc.load_gather(tblT, [kstride + (j * _CHUNK + c)])
      a = plsc.load_gather(cbuf, [av])
      b = plsc.load_gather(cbuf, [av + 1])
      packed = plsc.pack(a, b, format=plsc.PackFormat.INTERLEAVED)
      obuf[pl.ds((j * _CHUNK + c) * _DIM, _DIM)] = packed
      return 0

    lax.fori_loop(0, _CHUNK, convert_one, 0)

  pltpu.sync_copy(obuf, out_hbm.at[pl.ds(base * _DIM, _BPW * _DIM)])


@jax.jit
def _lookup(wt4, idx):
  mesh = plsc.VectorSubcoreMesh(core_axis_name="c", subcore_axis_name="s")
  f = pl.kernel(
      _sc_kernel,
      out_type=jax.ShapeDtypeStruct((_BATCH * _DIM,), jnp.bfloat16),
      mesh=mesh,
      scratch_types=[
          pltpu.VMEM((_BPW,), jnp.int32),
          pltpu.VMEM((_BPW,), jnp.int32),
          pltpu.VMEM((_L * _BPW,), jnp.int32),
          pltpu.VMEM((_BPW, _ROWW), jnp.float32),
          pltpu.VMEM((_CHUNK * _ROWW,), jnp.float32),
          pltpu.VMEM((_BPW * _DIM,), jnp.bfloat16),
          pltpu.VMEM_SHARED((_NS * _CHUNK * _ROWW,), jnp.float32),
          pltpu.SemaphoreType.DMA((_NCHUNKS,)),
          pltpu.SemaphoreType.DMA,
      ],
      compiler_params=pltpu.CompilerParams(needs_layout_passes=False),
  )
  return f(wt4, idx)


_NROWS = 1000000 * _DIM // _ROWW
_CI = 512
_GRID = -(-1000000 // _CI)


def _tc_detile(wt_ref, o_ref):
  x = wt_ref[...]
  eye = (lax.iota(jnp.int32, _DIM)[:, None] ==
         lax.iota(jnp.int32, _DIM)[None, :]).astype(jnp.float32)
  y = lax.dot_general(x, eye, (((0,), (0,)), ((), ())),
                      preferred_element_type=jnp.float32)
  y = y.reshape(_CI // 4, 4, _DIM)
  for k in range(4):
    o_ref[:, _DIM * k:_DIM * (k + 1)] = y[:, k, :]


@jax.jit
def _detile(wt):
  return pl.pallas_call(
      _tc_detile,
      grid=(_GRID,),
      in_specs=[pl.BlockSpec((_DIM, _CI), lambda ci: (0, ci))],
      out_specs=pl.BlockSpec((_CI // 4, _ROWW), lambda ci: (ci, 0)),
      out_shape=jax.ShapeDtypeStruct((_NROWS, _ROWW), jnp.float32),
  )(wt)


def kernel(inputs, weights):
  idx = inputs.astype(jnp.int32)
  wt4 = _detile(weights.T)
  out = _lookup(wt4, idx)
  return out.reshape(_BATCH, _DIM)


# speedup vs baseline: 2.6503x; 2.6503x over previous
import functools

import jax
import jax.numpy as jnp
from jax import lax
from jax.experimental import pallas as pl
from jax.experimental.pallas import tpu as pltpu
from jax.experimental.pallas import tpu_sc as plsc

_DIM = 32
_BATCH = 16384
_NC = 2
_NS = 16
_L = 16
_NW = _NC * _NS
_BPW = _BATCH // _NW
_CHUNK = 128
_NCHUNKS = _BPW // _CHUNK


def _sc_kernel(table_hbm, idx_hbm, out_hbm, idx_v, rows_v, out_v, sems):
  wid = lax.axis_index("s") * _NC + lax.axis_index("c")

  pltpu.sync_copy(idx_hbm.at[wid], idx_v)

  copies = []
  for j in range(_NCHUNKS):
    copies.append(
        pltpu.async_copy(
            table_hbm.at[idx_v.at[j]],
            rows_v.at[pl.ds(j * _CHUNK, _CHUNK)],
            sems.at[j],
        )
    )

  even = lax.iota(jnp.int32, _L) * 2
  odd = even + 1

  for j in range(_NCHUNKS):
    copies[j].wait()

    def convert_row(r, _):
      base = r * _DIM
      row = jnp.full((_L,), r, jnp.int32)
      a = plsc.load_gather(rows_v, [row, even])
      b = plsc.load_gather(rows_v, [row, odd])
      packed = plsc.pack(a, b, format=plsc.PackFormat.INTERLEAVED)
      out_v[pl.ds(base, _DIM)] = packed
      return 0

    lax.fori_loop(j * _CHUNK, (j + 1) * _CHUNK, convert_row, 0)

  pltpu.sync_copy(out_v, out_hbm.at[pl.ds(wid * _BPW * _DIM, _BPW * _DIM)])


@jax.jit
def _lookup(table, idx):
  mesh = plsc.VectorSubcoreMesh(core_axis_name="c", subcore_axis_name="s")
  f = pl.kernel(
      _sc_kernel,
      out_type=jax.ShapeDtypeStruct((_BATCH * _DIM,), jnp.bfloat16),
      mesh=mesh,
      scratch_types=[
          pltpu.VMEM((_NCHUNKS, _CHUNK), jnp.int32),
          pltpu.VMEM((_BPW, _DIM), jnp.float32),
          pltpu.VMEM((_BPW * _DIM,), jnp.bfloat16),
          pltpu.SemaphoreType.DMA((_NCHUNKS,)),
      ],
      compiler_params=pltpu.CompilerParams(
          needs_layout_passes=False, use_tc_tiling_on_sc=False
      ),
  )
  return f(table, idx)


def kernel(inputs, weights):
  idx = inputs.astype(jnp.int32).reshape(_NW, _NCHUNKS, _CHUNK)
  out = _lookup(weights, idx)
  return out.reshape(_BATCH, _DIM)
